# Initial kernel scaffold; baseline (speedup 1.0000x reference)
import jax
import jax.numpy as jnp
from jax.experimental import pallas as pl


def kernel(x, conv_w, gamma, beta):
    raise NotImplementedError("write your pallas kernel here")


# speedup vs baseline: 7.5450x; 7.5450x over previous
import functools

import jax
import jax.numpy as jnp
from jax.experimental import pallas as pl
from jax.experimental.pallas import tpu as pltpu

_BN_EPS = 1e-5
_CIN = 4
_COUT = 9
_K = 3


def _conv_channels(x_ref, w_ref, b):
    h, w = x_ref.shape[2], x_ref.shape[3]
    zrow = jnp.zeros((1, w), jnp.float32)
    zcol = jnp.zeros((h, 1), jnp.float32)
    acc = [None] * _COUT
    for ci in range(_CIN):
        x0 = x_ref[b, ci]
        rows = (
            jnp.concatenate([zrow, x0[: h - 1, :]], axis=0),
            x0,
            jnp.concatenate([x0[1:, :], zrow], axis=0),
        )
        for ki in range(_K):
            r = rows[ki]
            cols = (
                jnp.concatenate([zcol, r[:, : w - 1]], axis=1),
                r,
                jnp.concatenate([r[:, 1:], zcol], axis=1),
            )
            for kj in range(_K):
                sl = cols[kj]
                for c in range(_COUT):
                    wv = w_ref[((c * _CIN + ci) * _K + ki) * _K + kj]
                    t = sl * wv
                    acc[c] = t if acc[c] is None else acc[c] + t
    return acc


def _conv_kernel(x_ref, w_ref, y_ref, sum_ref, sq_ref, *, ib):
    for b in range(ib):
        acc = _conv_channels(x_ref, w_ref, b)
        for c in range(_COUT):
            y = acc[c]
            y_ref[b, c] = y
            sum_ref[b, c, :] = jnp.sum(y, axis=0)
            sq_ref[b, c, :] = jnp.sum(y * y, axis=0)


def _finish_kernel(y_ref, sc_ref, sh_ref, o_ref, *, ib):
    for b in range(ib):
        z = [y_ref[b, c] * sc_ref[c] + sh_ref[c] for c in range(_COUT)]
        m = z[0]
        for c in range(1, _COUT):
            m = jnp.maximum(m, z[c])
        e = [jnp.exp(z[c] - m) for c in range(_COUT)]
        d = e[0]
        for c in range(1, _COUT):
            d = d + e[c]
        r = pl.reciprocal(d, approx=True)
        for c in range(_COUT):
            o_ref[b, 0, c] = e[c] * r


@jax.jit
def _adcnn(x, conv_w, gamma, beta):
    n, c_in, h, w = x.shape
    x = x.astype(jnp.float32)
    wf = conv_w.reshape(_COUT * _CIN * _K * _K).astype(jnp.float32)

    ib = 1
    grid = (n // ib,)

    ybuf, sums, sqs = pl.pallas_call(
        functools.partial(_conv_kernel, ib=ib),
        out_shape=(
            jax.ShapeDtypeStruct((n, _COUT, h, w), jnp.float32),
            jax.ShapeDtypeStruct((n, _COUT, w), jnp.float32),
            jax.ShapeDtypeStruct((n, _COUT, w), jnp.float32),
        ),
        grid=grid,
        in_specs=[
            pl.BlockSpec((ib, c_in, h, w), lambda i: (i, 0, 0, 0)),
            pl.BlockSpec(memory_space=pltpu.SMEM),
        ],
        out_specs=(
            pl.BlockSpec((ib, _COUT, h, w), lambda i: (i, 0, 0, 0)),
            pl.BlockSpec((ib, _COUT, w), lambda i: (i, 0, 0)),
            pl.BlockSpec((ib, _COUT, w), lambda i: (i, 0, 0)),
        ),
        compiler_params=pltpu.CompilerParams(
            dimension_semantics=("parallel",)),
    )(x, wf)

    m_dim = n * h * w
    mean = jnp.sum(sums, axis=(0, 2)) / m_dim
    ex2 = jnp.sum(sqs, axis=(0, 2)) / m_dim
    var = jnp.maximum(ex2 - mean * mean, 0.0)
    scale = gamma.astype(jnp.float32) * jax.lax.rsqrt(var + _BN_EPS)
    shift = beta.astype(jnp.float32) - mean * scale

    ib2 = 4 if n % 4 == 0 else 1
    out5 = pl.pallas_call(
        functools.partial(_finish_kernel, ib=ib2),
        out_shape=jax.ShapeDtypeStruct((n, 1, _COUT, h, w), jnp.float32),
        grid=(n // ib2,),
        in_specs=[
            pl.BlockSpec((ib2, _COUT, h, w), lambda i: (i, 0, 0, 0)),
            pl.BlockSpec(memory_space=pltpu.SMEM),
            pl.BlockSpec(memory_space=pltpu.SMEM),
        ],
        out_specs=pl.BlockSpec((ib2, 1, _COUT, h, w),
                               lambda i: (i, 0, 0, 0, 0)),
        compiler_params=pltpu.CompilerParams(
            dimension_semantics=("parallel",)),
    )(ybuf, scale, shift)

    return out5.reshape(n, 1, _COUT, h * w)


def kernel(x, conv_w, gamma, beta):
    return _adcnn(x, conv_w, gamma, beta)


# speedup vs baseline: 7.9215x; 1.0499x over previous
import functools

import jax
import jax.numpy as jnp
from jax.experimental import pallas as pl
from jax.experimental.pallas import tpu as pltpu

_BN_EPS = 1e-5
_CIN = 4
_COUT = 9
_K = 3


def _conv_channels(x_ref, w_ref, b):
    h, w = x_ref.shape[2], x_ref.shape[3]
    zrow = jnp.zeros((1, w), jnp.float32)
    zcol = jnp.zeros((h, 1), jnp.float32)
    acc = [None] * _COUT
    for ci in range(_CIN):
        x0 = x_ref[b, ci]
        rows = (
            jnp.concatenate([zrow, x0[: h - 1, :]], axis=0),
            x0,
            jnp.concatenate([x0[1:, :], zrow], axis=0),
        )
        for ki in range(_K):
            r = rows[ki]
            cols = (
                jnp.concatenate([zcol, r[:, : w - 1]], axis=1),
                r,
                jnp.concatenate([r[:, 1:], zcol], axis=1),
            )
            for kj in range(_K):
                sl = cols[kj]
                for c in range(_COUT):
                    wv = w_ref[((c * _CIN + ci) * _K + ki) * _K + kj]
                    t = sl * wv
                    acc[c] = t if acc[c] is None else acc[c] + t
    return acc


def _conv_kernel(x_ref, w_ref, y_ref, sum_ref, sq_ref, *, ib):
    for b in range(ib):
        acc = _conv_channels(x_ref, w_ref, b)
        for c in range(_COUT):
            y = acc[c]
            y_ref[b, c] = y.astype(jnp.bfloat16)
            sum_ref[b, c, :] = jnp.sum(y, axis=0)
            sq_ref[b, c, :] = jnp.sum(y * y, axis=0)


def _finish_kernel(y_ref, sc_ref, sh_ref, o_ref, *, ib):
    for b in range(ib):
        z = [y_ref[b, c].astype(jnp.float32) * sc_ref[c] + sh_ref[c]
             for c in range(_COUT)]
        m = z[0]
        for c in range(1, _COUT):
            m = jnp.maximum(m, z[c])
        e = [jnp.exp(z[c] - m) for c in range(_COUT)]
        d = e[0]
        for c in range(1, _COUT):
            d = d + e[c]
        r = pl.reciprocal(d, approx=True)
        for c in range(_COUT):
            o_ref[b, 0, c] = e[c] * r


@jax.jit
def _adcnn(x, conv_w, gamma, beta):
    n, c_in, h, w = x.shape
    x = x.astype(jnp.float32)
    wf = conv_w.reshape(_COUT * _CIN * _K * _K).astype(jnp.float32)

    ib = 1
    grid = (n // ib,)

    ybuf, sums, sqs = pl.pallas_call(
        functools.partial(_conv_kernel, ib=ib),
        out_shape=(
            jax.ShapeDtypeStruct((n, _COUT, h, w), jnp.bfloat16),
            jax.ShapeDtypeStruct((n, _COUT, w), jnp.float32),
            jax.ShapeDtypeStruct((n, _COUT, w), jnp.float32),
        ),
        grid=grid,
        in_specs=[
            pl.BlockSpec((ib, c_in, h, w), lambda i: (i, 0, 0, 0)),
            pl.BlockSpec(memory_space=pltpu.SMEM),
        ],
        out_specs=(
            pl.BlockSpec((ib, _COUT, h, w), lambda i: (i, 0, 0, 0)),
            pl.BlockSpec((ib, _COUT, w), lambda i: (i, 0, 0)),
            pl.BlockSpec((ib, _COUT, w), lambda i: (i, 0, 0)),
        ),
        compiler_params=pltpu.CompilerParams(
            dimension_semantics=("parallel",)),
    )(x, wf)

    m_dim = n * h * w
    mean = jnp.sum(sums, axis=(0, 2)) / m_dim
    ex2 = jnp.sum(sqs, axis=(0, 2)) / m_dim
    var = jnp.maximum(ex2 - mean * mean, 0.0)
    scale = gamma.astype(jnp.float32) * jax.lax.rsqrt(var + _BN_EPS)
    shift = beta.astype(jnp.float32) - mean * scale

    ib2 = 8 if n % 8 == 0 else 1
    out5 = pl.pallas_call(
        functools.partial(_finish_kernel, ib=ib2),
        out_shape=jax.ShapeDtypeStruct((n, 1, _COUT, h, w), jnp.float32),
        grid=(n // ib2,),
        in_specs=[
            pl.BlockSpec((ib2, _COUT, h, w), lambda i: (i, 0, 0, 0)),
            pl.BlockSpec(memory_space=pltpu.SMEM),
            pl.BlockSpec(memory_space=pltpu.SMEM),
        ],
        out_specs=pl.BlockSpec((ib2, 1, _COUT, h, w),
                               lambda i: (i, 0, 0, 0, 0)),
        compiler_params=pltpu.CompilerParams(
            dimension_semantics=("parallel",)),
    )(ybuf, scale, shift)

    return out5.reshape(n, 1, _COUT, h * w)


def kernel(x, conv_w, gamma, beta):
    return _adcnn(x, conv_w, gamma, beta)
